# baseline (speedup 1.0000x reference)
import functools
import jax
import jax.numpy as jnp
from jax import lax
from jax.experimental import pallas as pl
from jax.experimental.pallas import tpu as pltpu
from jax.experimental.pallas import tpu_sc as plsc

_CHUNK = 32768
_LANES = 16


def kernel(inputs):
    m, n = inputs.shape
    total = m * n
    info = plsc.get_sparse_core_info()
    nw = info.num_cores * info.num_subcores
    per_w = total // nw
    chunks = per_w // _CHUNK
    assert per_w % _CHUNK == 0

    mesh = plsc.VectorSubcoreMesh(core_axis_name="c", subcore_axis_name="s")

    @functools.partial(
        pl.kernel,
        mesh=mesh,
        out_type=jax.ShapeDtypeStruct((total,), jnp.float32),
        scratch_types=[pltpu.VMEM((_CHUNK,), jnp.float32)],
    )
    def _sc_shift(x_hbm, o_hbm, buf):
        wid = lax.axis_index("s") * info.num_cores + lax.axis_index("c")
        base = wid * per_w

        def chunk_body(ci, _):
            off = base + ci * _CHUNK
            pltpu.sync_copy(x_hbm.at[pl.ds(off, _CHUNK)], buf)

            def vec_body(vi, _):
                v = buf[pl.ds(vi * _LANES, _LANES)]
                buf[pl.ds(vi * _LANES, _LANES)] = jnp.where(
                    v > 0.0, jnp.full((_LANES,), 1.0, jnp.float32),
                    jnp.full((_LANES,), -1.0, jnp.float32))
                return 0

            lax.fori_loop(0, _CHUNK // _LANES, vec_body, 0)
            pltpu.sync_copy(buf, o_hbm.at[pl.ds(off, _CHUNK)])
            return 0

        lax.fori_loop(0, chunks, chunk_body, 0)

    flat = inputs.reshape(total)
    out = _sc_shift(flat)
    return out.reshape(m, n)


# speedup vs baseline: 1.9195x; 1.9195x over previous
import functools
import jax
import jax.numpy as jnp
from jax import lax
from jax.experimental import pallas as pl
from jax.experimental.pallas import tpu as pltpu
from jax.experimental.pallas import tpu_sc as plsc

_CHUNK = 16384
_LANES = 16
_UNROLL = 8


def kernel(inputs):
    m, n = inputs.shape
    total = m * n
    info = plsc.get_sparse_core_info()
    nw = info.num_cores * info.num_subcores
    per_w = total // nw
    nchunks = per_w // _CHUNK
    assert per_w % _CHUNK == 0 and nchunks % 2 == 0 and nchunks >= 4

    mesh = plsc.VectorSubcoreMesh(core_axis_name="c", subcore_axis_name="s")

    @functools.partial(
        pl.kernel,
        mesh=mesh,
        out_type=jax.ShapeDtypeStruct((total,), jnp.float32),
        scratch_types=[
            pltpu.VMEM((_CHUNK,), jnp.float32),
            pltpu.VMEM((_CHUNK,), jnp.float32),
            pltpu.VMEM((_CHUNK,), jnp.float32),
            pltpu.VMEM((_CHUNK,), jnp.float32),
            pltpu.SemaphoreType.DMA,
            pltpu.SemaphoreType.DMA,
            pltpu.SemaphoreType.DMA,
            pltpu.SemaphoreType.DMA,
        ],
    )
    def _sc_shift(x_hbm, o_hbm, i0, i1, o0, o1, si0, si1, so0, so1):
        wid = lax.axis_index("s") * info.num_cores + lax.axis_index("c")
        base = wid * per_w
        ibufs, obufs = (i0, i1), (o0, o1)
        isems, osems = (si0, si1), (so0, so1)

        def start_in(ci, b):
            pltpu.async_copy(x_hbm.at[pl.ds(base + ci * _CHUNK, _CHUNK)],
                             ibufs[b], isems[b])

        def wait_in(ci, b):
            pltpu.make_async_copy(x_hbm.at[pl.ds(base + ci * _CHUNK, _CHUNK)],
                                  ibufs[b], isems[b]).wait()

        def start_out(ci, b):
            pltpu.async_copy(obufs[b],
                             o_hbm.at[pl.ds(base + ci * _CHUNK, _CHUNK)],
                             osems[b])

        def wait_out(ci, b):
            pltpu.make_async_copy(obufs[b],
                                  o_hbm.at[pl.ds(base + ci * _CHUNK, _CHUNK)],
                                  osems[b]).wait()

        def compute(b):
            src, dst = ibufs[b], obufs[b]

            def vec_body(vi, _):
                for u in range(_UNROLL):
                    sl = pl.ds((vi * _UNROLL + u) * _LANES, _LANES)
                    v = src[sl]
                    dst[sl] = jnp.where(
                        v > 0.0, jnp.full((_LANES,), 1.0, jnp.float32),
                        jnp.full((_LANES,), -1.0, jnp.float32))
                return 0

            lax.fori_loop(0, _CHUNK // (_LANES * _UNROLL), vec_body, 0)

        start_in(0, 0)
        start_in(1, 1)
        for b in range(2):
            wait_in(b, b)
            compute(b)
            start_out(b, b)
            start_in(b + 2, b)

        def pair_body(k, _):
            ci0 = 2 + k * 2
            for b in range(2):
                ci = ci0 + b
                wait_in(ci, b)
                wait_out(ci - 2, b)
                compute(b)
                start_out(ci, b)

                @pl.when(ci + 2 < nchunks)
                def _():
                    start_in(ci + 2, b)

            return 0

        lax.fori_loop(0, (nchunks - 2) // 2, pair_body, 0)

        wait_out(nchunks - 2, 0)
        wait_out(nchunks - 1, 1)

    flat = inputs.reshape(total)
    out = _sc_shift(flat)
    return out.reshape(m, n)


# speedup vs baseline: 1.9232x; 1.0019x over previous
import functools
import jax
import jax.numpy as jnp
from jax import lax
from jax.experimental import pallas as pl
from jax.experimental.pallas import tpu as pltpu
from jax.experimental.pallas import tpu_sc as plsc

_CHUNK = 32768


def kernel(inputs):
    m, n = inputs.shape
    total = m * n
    info = plsc.get_sparse_core_info()
    nw = info.num_cores * info.num_subcores
    per_w = total // nw
    nchunks = per_w // _CHUNK
    assert per_w % _CHUNK == 0 and nchunks % 2 == 0

    mesh = plsc.VectorSubcoreMesh(core_axis_name="c", subcore_axis_name="s")

    @functools.partial(
        pl.kernel,
        mesh=mesh,
        out_type=jax.ShapeDtypeStruct((total,), jnp.float32),
        scratch_types=[
            pltpu.VMEM((_CHUNK,), jnp.float32),
            pltpu.VMEM((_CHUNK,), jnp.float32),
            pltpu.SemaphoreType.DMA,
            pltpu.SemaphoreType.DMA,
            pltpu.SemaphoreType.DMA,
            pltpu.SemaphoreType.DMA,
        ],
    )
    def _probe(x_hbm, o_hbm, b0, b1, si0, si1, so0, so1):
        wid = lax.axis_index("s") * info.num_cores + lax.axis_index("c")
        base = wid * per_w
        bufs = (b0, b1)
        isems, osems = (si0, si1), (so0, so1)

        def start_in(ci, b):
            pltpu.async_copy(x_hbm.at[pl.ds(base + ci * _CHUNK, _CHUNK)],
                             bufs[b], isems[b])

        def wait_in(ci, b):
            pltpu.make_async_copy(x_hbm.at[pl.ds(base + ci * _CHUNK, _CHUNK)],
                                  bufs[b], isems[b]).wait()

        def start_out(ci, b):
            pltpu.async_copy(bufs[b],
                             o_hbm.at[pl.ds(base + ci * _CHUNK, _CHUNK)],
                             osems[b])

        def wait_out(ci, b):
            pltpu.make_async_copy(bufs[b],
                                  o_hbm.at[pl.ds(base + ci * _CHUNK, _CHUNK)],
                                  osems[b]).wait()

        start_in(0, 0)
        start_in(1, 1)
        for b in range(2):
            wait_in(b, b)
            start_out(b, b)

        def pair_body(k, _):
            ci0 = 2 + k * 2
            for b in range(2):
                ci = ci0 + b
                wait_out(ci - 2, b)
                start_in(ci, b)
                wait_in(ci, b)
                start_out(ci, b)
            return 0

        lax.fori_loop(0, (nchunks - 2) // 2, pair_body, 0)
        wait_out(nchunks - 2, 0)
        wait_out(nchunks - 1, 1)

    flat = inputs.reshape(total)
    out = _probe(flat)
    return out.reshape(m, n)


# speedup vs baseline: 7.9377x; 4.1273x over previous
import jax
import jax.numpy as jnp
from jax.experimental import pallas as pl


def _shift_kernel(x_ref, o_ref):
    o_ref[...] = jnp.where(x_ref[...] > 0, 1.0, -1.0).astype(jnp.float32)


def kernel(inputs):
    m, n = inputs.shape
    bm = 4096 if m % 4096 == 0 else m
    grid = (m // bm,)
    return pl.pallas_call(
        _shift_kernel,
        grid=grid,
        in_specs=[pl.BlockSpec((bm, n), lambda i: (i, 0))],
        out_specs=pl.BlockSpec((bm, n), lambda i: (i, 0)),
        out_shape=jax.ShapeDtypeStruct((m, n), jnp.float32),
    )(inputs)
